# baseline (speedup 1.0000x reference)
import functools

import jax
import jax.numpy as jnp
from jax import lax
from jax.experimental import pallas as pl
from jax.experimental.pallas import tpu as pltpu
from jax.experimental.pallas import tpu_sc as plsc

N_ROWS = 1000000
EMB = 16
BATCH = 16384
PACK = 8

NUM_CORES = 2
NUM_SUBCORES = 16
NUM_WORKERS = NUM_CORES * NUM_SUBCORES
BPW = BATCH // NUM_WORKERS
GCHUNK = 128
NGCHUNK = BPW // GCHUNK
BLOCKS_PER_CHUNK = GCHUNK // EMB


def _bpr_body(ud8_hbm, ucb_hbm, pd8_hbm, pcb_hbm, nd8_hbm, ncb_hbm,
              ut_hbm, it_hbm,
              rui_hbm, ruj_hbm, loss_hbm,
              ud8_v, ucb_v, pd8_v, pcb_v, nd8_v, ncb_v,
              u_b0, u_b1, p_b0, p_b1, n_b0, n_b1,
              rui_v, ruj_v, loss_v, sem0, sem1):
    wid = lax.axis_index("s") * NUM_CORES + lax.axis_index("c")
    base = wid * BPW

    pltpu.sync_copy(ud8_hbm.at[pl.ds(base, BPW)], ud8_v)
    pltpu.sync_copy(ucb_hbm.at[pl.ds(base, BPW)], ucb_v)
    pltpu.sync_copy(pd8_hbm.at[pl.ds(base, BPW)], pd8_v)
    pltpu.sync_copy(pcb_hbm.at[pl.ds(base, BPW)], pcb_v)
    pltpu.sync_copy(nd8_hbm.at[pl.ds(base, BPW)], nd8_v)
    pltpu.sync_copy(ncb_hbm.at[pl.ds(base, BPW)], ncb_v)

    bufs = ((u_b0, p_b0, n_b0, sem0), (u_b1, p_b1, n_b1, sem1))

    def fire(c, slot):
        ub, pb, nb, sem = bufs[slot]
        s = pl.ds(c * GCHUNK, GCHUNK)
        return (pltpu.async_copy(ut_hbm.at[ud8_v.at[s]], ub, sem),
                pltpu.async_copy(it_hbm.at[pd8_v.at[s]], pb, sem),
                pltpu.async_copy(it_hbm.at[nd8_v.at[s]], nb, sem))

    def compute(c, slot, loss_acc):
        ub, pb, nb, _ = bufs[slot]

        def block(b, loss_acc):
            rows = b * EMB + lax.iota(jnp.int32, EMB)
            off = pl.ds(c * GCHUNK + b * EMB, EMB)
            cbu = ucb_v[off]
            cbp = pcb_v[off]
            cbn = ncb_v[off]
            acc_ui = jnp.zeros((EMB,), jnp.float32)
            acc_uj = jnp.zeros((EMB,), jnp.float32)
            for e in range(EMB):
                u = plsc.load_gather(ub, [rows, cbu + e])
                p = plsc.load_gather(pb, [rows, cbp + e])
                n = plsc.load_gather(nb, [rows, cbn + e])
                acc_ui = acc_ui + u * p
                acc_uj = acc_uj + u * n
                loss_acc = loss_acc + (u * u + p * p + n * n)
            rui_v[off] = acc_ui
            ruj_v[off] = acc_uj
            return loss_acc

        return lax.fori_loop(0, BLOCKS_PER_CHUNK, block, loss_acc)

    loss_acc = jnp.zeros((EMB,), jnp.float32)
    h0 = fire(0, 0)
    h1 = fire(1, 1)
    for cp in h0:
        cp.wait()
    loss_acc = compute(0, 0, loss_acc)
    h2 = fire(2, 0)
    for cp in h1:
        cp.wait()
    loss_acc = compute(1, 1, loss_acc)
    h3 = fire(3, 1)
    for cp in h2:
        cp.wait()
    loss_acc = compute(2, 0, loss_acc)
    for cp in h3:
        cp.wait()
    loss_acc = compute(3, 1, loss_acc)
    loss_v[...] = loss_acc

    pltpu.sync_copy(rui_v, rui_hbm.at[pl.ds(base, BPW)])
    pltpu.sync_copy(ruj_v, ruj_hbm.at[pl.ds(base, BPW)])
    pltpu.sync_copy(loss_v, loss_hbm.at[wid])


@jax.jit
def _bpr_sc(ud8, ucb, pd8, pcb, nd8, ncb, ut, it):
    mesh = plsc.VectorSubcoreMesh(core_axis_name="c", subcore_axis_name="s")
    kern = functools.partial(
        pl.kernel,
        mesh=mesh,
        compiler_params=pltpu.CompilerParams(needs_layout_passes=False),
        out_type=[
            jax.ShapeDtypeStruct((BATCH,), jnp.float32),
            jax.ShapeDtypeStruct((BATCH,), jnp.float32),
            jax.ShapeDtypeStruct((NUM_WORKERS, EMB), jnp.float32),
        ],
        scratch_types=[
            pltpu.VMEM((BPW,), jnp.int32),
            pltpu.VMEM((BPW,), jnp.int32),
            pltpu.VMEM((BPW,), jnp.int32),
            pltpu.VMEM((BPW,), jnp.int32),
            pltpu.VMEM((BPW,), jnp.int32),
            pltpu.VMEM((BPW,), jnp.int32),
            pltpu.VMEM((GCHUNK, PACK * EMB), jnp.float32),
            pltpu.VMEM((GCHUNK, PACK * EMB), jnp.float32),
            pltpu.VMEM((GCHUNK, PACK * EMB), jnp.float32),
            pltpu.VMEM((GCHUNK, PACK * EMB), jnp.float32),
            pltpu.VMEM((GCHUNK, PACK * EMB), jnp.float32),
            pltpu.VMEM((GCHUNK, PACK * EMB), jnp.float32),
            pltpu.VMEM((BPW,), jnp.float32),
            pltpu.VMEM((BPW,), jnp.float32),
            pltpu.VMEM((EMB,), jnp.float32),
            pltpu.SemaphoreType.DMA,
            pltpu.SemaphoreType.DMA,
        ],
    )(_bpr_body)
    return kern(ud8, ucb, pd8, pcb, nd8, ncb, ut, it)


def kernel(users, pos_items, neg_items, user_emb, item_emb):
    users = users.astype(jnp.int32)
    pos_items = pos_items.astype(jnp.int32)
    neg_items = neg_items.astype(jnp.int32)
    ud8 = users >> 3
    ucb = (users & 7) << 4
    pd8 = pos_items >> 3
    pcb = (pos_items & 7) << 4
    nd8 = neg_items >> 3
    ncb = (neg_items & 7) << 4
    ut = user_emb.reshape(N_ROWS // PACK, PACK * EMB)
    it = item_emb.reshape(N_ROWS // PACK, PACK * EMB)
    rui, ruj, loss_parts = _bpr_sc(ud8, ucb, pd8, pcb, nd8, ncb, ut, it)
    return (rui.reshape(BATCH, 1), ruj.reshape(BATCH, 1),
            jnp.sum(loss_parts))


# speedup vs baseline: 4.6577x; 4.6577x over previous
import functools

import jax
import jax.numpy as jnp
from jax import lax
from jax.experimental import pallas as pl
from jax.experimental.pallas import tpu as pltpu
from jax.experimental.pallas import tpu_sc as plsc

N_ROWS = 1000000
EMB = 16
BATCH = 16384
LANES = 128

NUM_CORES = 2
NUM_SUBCORES = 16
NUM_WORKERS = NUM_CORES * NUM_SUBCORES
BPW = BATCH // NUM_WORKERS
GROUP = 16
NGROUPS = BPW // GROUP


def _bpr_body(uidx_hbm, pidx_hbm, nidx_hbm, ut_hbm, it_hbm,
              rui_hbm, ruj_hbm, loss_hbm,
              uidx_v, pidx_v, nidx_v,
              u_s, p_s, n_s,
              rui_v, ruj_v, loss_v, sem):
    wid = lax.axis_index("s") * NUM_CORES + lax.axis_index("c")
    base = wid * BPW

    pltpu.sync_copy(uidx_hbm.at[pl.ds(base, BPW)], uidx_v)
    pltpu.sync_copy(pidx_hbm.at[pl.ds(base, BPW)], pidx_v)
    pltpu.sync_copy(nidx_hbm.at[pl.ds(base, BPW)], nidx_v)

    def group(g, loss_acc):
        goff = pl.ds(g * GROUP, GROUP)
        iv_u = uidx_v[goff]
        iv_p = pidx_v[goff]
        iv_n = nidx_v[goff]
        blk_u = iv_u & ~(LANES - 1)
        blk_p = iv_p & ~(LANES - 1)
        blk_n = iv_n & ~(LANES - 1)
        handles = []
        for i in range(GROUP):
            dst = pl.ds(i * EMB, EMB)
            bu = pl.multiple_of(blk_u[i], LANES)
            bp = pl.multiple_of(blk_p[i], LANES)
            bn = pl.multiple_of(blk_n[i], LANES)
            handles.append(pltpu.async_copy(
                ut_hbm.at[:, pl.ds(bu, LANES)], u_s.at[dst], sem))
            handles.append(pltpu.async_copy(
                it_hbm.at[:, pl.ds(bp, LANES)], p_s.at[dst], sem))
            handles.append(pltpu.async_copy(
                it_hbm.at[:, pl.ds(bn, LANES)], n_s.at[dst], sem))
        for cp in handles:
            cp.wait()

        item_rows = lax.iota(jnp.int32, GROUP) * EMB
        lane_u = iv_u & (LANES - 1)
        lane_p = iv_p & (LANES - 1)
        lane_n = iv_n & (LANES - 1)

        acc_ui = jnp.zeros((GROUP,), jnp.float32)
        acc_uj = jnp.zeros((GROUP,), jnp.float32)
        for e in range(EMB):
            rows = item_rows + e
            u = plsc.load_gather(u_s, [rows, lane_u])
            p = plsc.load_gather(p_s, [rows, lane_p])
            n = plsc.load_gather(n_s, [rows, lane_n])
            acc_ui = acc_ui + u * p
            acc_uj = acc_uj + u * n
            loss_acc = loss_acc + (u * u + p * p + n * n)
        rui_v[goff] = acc_ui
        ruj_v[goff] = acc_uj
        return loss_acc

    loss_acc = lax.fori_loop(0, NGROUPS, group, jnp.zeros((GROUP,), jnp.float32))
    loss_v[...] = loss_acc

    pltpu.sync_copy(rui_v, rui_hbm.at[pl.ds(base, BPW)])
    pltpu.sync_copy(ruj_v, ruj_hbm.at[pl.ds(base, BPW)])
    pltpu.sync_copy(loss_v, loss_hbm.at[wid])


@jax.jit
def _bpr_sc(uidx, pidx, nidx, ut, it):
    mesh = plsc.VectorSubcoreMesh(core_axis_name="c", subcore_axis_name="s")
    kern = functools.partial(
        pl.kernel,
        mesh=mesh,
        compiler_params=pltpu.CompilerParams(needs_layout_passes=False),
        out_type=[
            jax.ShapeDtypeStruct((BATCH,), jnp.float32),
            jax.ShapeDtypeStruct((BATCH,), jnp.float32),
            jax.ShapeDtypeStruct((NUM_WORKERS, EMB), jnp.float32),
        ],
        scratch_types=[
            pltpu.VMEM((BPW,), jnp.int32),
            pltpu.VMEM((BPW,), jnp.int32),
            pltpu.VMEM((BPW,), jnp.int32),
            pltpu.VMEM((GROUP * EMB, LANES), jnp.float32),
            pltpu.VMEM((GROUP * EMB, LANES), jnp.float32),
            pltpu.VMEM((GROUP * EMB, LANES), jnp.float32),
            pltpu.VMEM((BPW,), jnp.float32),
            pltpu.VMEM((BPW,), jnp.float32),
            pltpu.VMEM((EMB,), jnp.float32),
            pltpu.SemaphoreType.DMA,
        ],
    )(_bpr_body)
    return kern(uidx, pidx, nidx, ut, it)


def kernel(users, pos_items, neg_items, user_emb, item_emb):
    users = users.astype(jnp.int32)
    pos_items = pos_items.astype(jnp.int32)
    neg_items = neg_items.astype(jnp.int32)
    ut = user_emb.T
    it = item_emb.T
    rui, ruj, loss_parts = _bpr_sc(users, pos_items, neg_items, ut, it)
    return (rui.reshape(BATCH, 1), ruj.reshape(BATCH, 1),
            jnp.sum(loss_parts))
